# baseline (speedup 1.0000x reference)
import jax
import jax.numpy as jnp
from jax.experimental import pallas as pl
from jax.experimental.pallas import tpu as pltpu

N = 8192
M = 8192
K = 128
KA = K + 2
LANE = 128
SUB = 8
CH = 16
CW = M // CH


def _aug_x(b):
    g = 0.5 * jnp.sum(b * b, axis=1, keepdims=True)
    neg1 = jnp.full_like(g, -1.0)
    return jnp.concatenate([b, neg1, g], axis=1).astype(jnp.bfloat16)


def _aug_y(b):
    h = 0.5 * jnp.sum(b * b, axis=1, keepdims=True)
    neg1 = jnp.full_like(h, -1.0)
    return jnp.concatenate([b, h, neg1], axis=1).astype(jnp.bfloat16)


def _chamfer_kernel(x_ref, y_ref, out_ref):
    xa = _aug_x(x_ref[...])
    ya = _aug_y(y_ref[...])

    pr = None
    pcs = []
    for s in range(CH):
        ps = jax.lax.dot_general(
            xa, ya[s * CW:(s + 1) * CW, :], (((1,), (1,)), ((), ())),
            preferred_element_type=jnp.float32,
        )
        w = ps
        while w.shape[1] > LANE:
            h = w.shape[1] // 2
            w = jnp.maximum(w[:, :h], w[:, h:])
        pr = w if pr is None else jnp.maximum(pr, w)
        v = ps
        while v.shape[0] > SUB:
            h = v.shape[0] // 2
            v = jnp.maximum(v[:h, :], v[h:, :])
        pcs.append(v)
    pc = jnp.concatenate(pcs, axis=1)

    rm = jnp.max(pr, axis=1, keepdims=True)
    d_xy = jnp.maximum(-2.0 * rm, 0.0)
    cm = jnp.max(pc, axis=0, keepdims=True)
    d_yx = jnp.maximum(-2.0 * cm, 0.0)
    out_ref[...] = (jnp.sum(d_xy, keepdims=True)
                    + jnp.sum(d_yx, keepdims=True))


def kernel(x, y):
    out = pl.pallas_call(
        _chamfer_kernel,
        out_shape=jax.ShapeDtypeStruct((1, 1), jnp.float32),
    )(x, y)
    return out[0, 0]


# speedup vs baseline: 1.0512x; 1.0512x over previous
import jax
import jax.numpy as jnp
from jax.experimental import pallas as pl
from jax.experimental.pallas import tpu as pltpu

N = 8192
M = 8192
K = 128
KA = K + 2
LANE = 128
SUB = 8
NJ = 4
BJ = M // NJ
CH = 8
CW = BJ // CH


def _aug_x(b):
    g = 0.5 * jnp.sum(b * b, axis=1, keepdims=True)
    neg1 = jnp.full_like(g, -1.0)
    return jnp.concatenate([b, neg1, g], axis=1).astype(jnp.bfloat16)


def _aug_y(b):
    h = 0.5 * jnp.sum(b * b, axis=1, keepdims=True)
    neg1 = jnp.full_like(h, -1.0)
    return jnp.concatenate([b, h, neg1], axis=1).astype(jnp.bfloat16)


def _chamfer_step(x_ref, y_ref, out_ref, xa_s, rowacc):
    j = pl.program_id(0)

    @pl.when(j == 0)
    def _():
        xa_s[...] = _aug_x(x_ref[...])
        out_ref[...] = jnp.zeros((1, 1), jnp.float32)

    ya = _aug_y(y_ref[...])

    pr = None
    pcs = []
    for s in range(CH):
        ps = jax.lax.dot_general(
            xa_s[...], ya[s * CW:(s + 1) * CW, :], (((1,), (1,)), ((), ())),
            preferred_element_type=jnp.float32,
        )
        w = ps
        while w.shape[1] > LANE:
            h = w.shape[1] // 2
            w = jnp.maximum(w[:, :h], w[:, h:])
        pr = w if pr is None else jnp.maximum(pr, w)
        v = ps
        while v.shape[0] > SUB:
            h = v.shape[0] // 2
            v = jnp.maximum(v[:h, :], v[h:, :])
        pcs.append(v)

    @pl.when(j == 0)
    def _():
        rowacc[...] = pr

    @pl.when(j > 0)
    def _():
        rowacc[...] = jnp.maximum(rowacc[...], pr)

    pc = jnp.concatenate(pcs, axis=1)
    cm = jnp.max(pc, axis=0, keepdims=True)
    d_yx = jnp.maximum(-2.0 * cm, 0.0)
    out_ref[...] += jnp.sum(d_yx, keepdims=True)

    @pl.when(j == NJ - 1)
    def _():
        rm = jnp.max(rowacc[...], axis=1, keepdims=True)
        d_xy = jnp.maximum(-2.0 * rm, 0.0)
        out_ref[...] += jnp.sum(d_xy, keepdims=True)


def kernel(x, y):
    out = pl.pallas_call(
        _chamfer_step,
        grid=(NJ,),
        in_specs=[
            pl.BlockSpec((N, K), lambda j: (0, 0)),
            pl.BlockSpec((BJ, K), lambda j: (j, 0)),
        ],
        out_specs=pl.BlockSpec((1, 1), lambda j: (0, 0)),
        out_shape=jax.ShapeDtypeStruct((1, 1), jnp.float32),
        scratch_shapes=[
            pltpu.VMEM((N, KA), jnp.bfloat16),
            pltpu.VMEM((N, LANE), jnp.float32),
        ],
        compiler_params=pltpu.CompilerParams(
            dimension_semantics=("arbitrary",),
        ),
    )(x, y)
    return out[0, 0]
